# Initial kernel scaffold; baseline (speedup 1.0000x reference)
import jax
import jax.numpy as jnp
from jax.experimental import pallas as pl


def kernel(x, adj, PvT, W_heads, a_heads, W_out, a_out):
    raise NotImplementedError("write your pallas kernel here")


# speedup vs baseline: 4451.9607x; 4451.9607x over previous
import functools

import jax
import jax.numpy as jnp
from jax.experimental import pallas as pl

_ALPHA = 0.2
_BR = 256


def _elu(z):
    return jnp.where(z > 0, z, jnp.exp(jnp.minimum(z, 0.0)) - 1.0)


def _prep_kernel(x_ref, xT_ref, Wall_ref, WallT_ref, Asrc_ref, AdstT_ref,
                 h_ref, u_ref, p_ref, vT_ref, qT_ref):
    h = jnp.dot(x_ref[...], Wall_ref[...], preferred_element_type=jnp.float32)
    h_ref[...] = h
    hT = jnp.dot(WallT_ref[...], xT_ref[...], preferred_element_type=jnp.float32)
    ls = jnp.dot(h, Asrc_ref[...], preferred_element_type=jnp.float32)
    u_ref[...] = jnp.exp(-ls)
    p_ref[...] = jnp.exp(-_ALPHA * ls)
    ldT = jnp.dot(AdstT_ref[...], hT, preferred_element_type=jnp.float32)
    vT_ref[...] = jnp.exp(-ldT)
    qT_ref[...] = jnp.exp(-_ALPHA * ldT)


def _layer1_kernel(nheads, nhid,
                   adj_ref, h_ref, u_ref, p_ref, vT_ref, qT_ref,
                   Wout_ref, a2s_ref, a2d_ref,
                   h2_ref, u2_ref, p2_ref, v2_ref, q2_ref):
    adj = adj_ref[...]
    outs = []
    for hd in range(nheads):
        uc = u_ref[:, hd:hd + 1]
        pc = p_ref[:, hd:hd + 1]
        vr = vT_ref[hd:hd + 1, :]
        qr = qT_ref[hd:hd + 1, :]
        e = jnp.minimum(uc * vr, pc * qr) * adj
        hh = h_ref[:, hd * nhid:(hd + 1) * nhid]
        hp = jnp.dot(e, hh, preferred_element_type=jnp.float32)
        rs = jnp.sum(e, axis=1, keepdims=True)
        outs.append(_elu(hp / rs))
    x1 = jnp.concatenate(outs, axis=1)
    h2 = jnp.dot(x1, Wout_ref[...], preferred_element_type=jnp.float32)
    h2_ref[...] = h2
    ls2 = jnp.dot(h2, a2s_ref[...], preferred_element_type=jnp.float32)
    ld2 = jnp.dot(h2, a2d_ref[...], preferred_element_type=jnp.float32)
    u2_ref[...] = jnp.exp(-ls2)
    p2_ref[...] = jnp.exp(-_ALPHA * ls2)
    v2_ref[...] = jnp.exp(-ld2)
    q2_ref[...] = jnp.exp(-_ALPHA * ld2)


def _layer2_kernel(nblk,
                   adj_ref, h2_ref, u2_ref, p2_ref, v2T_ref, q2T_ref,
                   PvT_ref, out_ref):
    i = pl.program_id(0)
    adj = adj_ref[...]
    e = jnp.minimum(u2_ref[...] * v2T_ref[...],
                    p2_ref[...] * q2T_ref[...]) * adj
    hp = jnp.dot(e, h2_ref[...], preferred_element_type=jnp.float32)
    rs = jnp.sum(e, axis=1, keepdims=True)
    x2 = _elu(hp / rs)
    contrib = jnp.dot(PvT_ref[...], x2, preferred_element_type=jnp.float32)

    @pl.when(i == 0)
    def _():
        out_ref[...] = contrib

    @pl.when(i > 0)
    def _():
        out_ref[...] += contrib

    @pl.when(i == nblk - 1)
    def _():
        z = out_ref[...]
        m = jnp.max(z, axis=1, keepdims=True)
        zs = z - m
        out_ref[...] = zs - jnp.log(jnp.sum(jnp.exp(zs), axis=1, keepdims=True))


def kernel(x, adj, PvT, W_heads, a_heads, W_out, a_out):
    f32 = jnp.float32
    n, nfeat = x.shape
    nheads, _, nhid = W_heads.shape
    nclass = W_out.shape[1]
    nv = PvT.shape[0]
    fcat = nheads * nhid
    br = _BR if n % _BR == 0 else n
    nblk = n // br

    Wall = jnp.transpose(W_heads, (1, 0, 2)).reshape(nfeat, fcat)
    WallT = Wall.T
    a_src = a_heads[:, 0, :nhid]
    a_dst = a_heads[:, 0, nhid:]
    eye = jnp.eye(nheads, dtype=f32)
    Asrc = (eye[:, None, :] * a_src[:, :, None]).reshape(fcat, nheads)
    AdstT = (eye[:, :, None] * a_dst[None, :, :]).reshape(nheads, fcat)
    a2s = a_out[0, :nclass].reshape(nclass, 1)
    a2d = a_out[0, nclass:].reshape(nclass, 1)
    xT = x.T

    h_all, u1, p1, v1T, q1T = pl.pallas_call(
        _prep_kernel,
        out_shape=[
            jax.ShapeDtypeStruct((n, fcat), f32),
            jax.ShapeDtypeStruct((n, nheads), f32),
            jax.ShapeDtypeStruct((n, nheads), f32),
            jax.ShapeDtypeStruct((nheads, n), f32),
            jax.ShapeDtypeStruct((nheads, n), f32),
        ],
    )(x, xT, Wall, WallT, Asrc, AdstT)

    h2, u2, p2, v2c, q2c = pl.pallas_call(
        functools.partial(_layer1_kernel, nheads, nhid),
        grid=(nblk,),
        in_specs=[
            pl.BlockSpec((br, n), lambda i: (i, 0)),
            pl.BlockSpec((n, fcat), lambda i: (0, 0)),
            pl.BlockSpec((br, nheads), lambda i: (i, 0)),
            pl.BlockSpec((br, nheads), lambda i: (i, 0)),
            pl.BlockSpec((nheads, n), lambda i: (0, 0)),
            pl.BlockSpec((nheads, n), lambda i: (0, 0)),
            pl.BlockSpec((fcat, nclass), lambda i: (0, 0)),
            pl.BlockSpec((nclass, 1), lambda i: (0, 0)),
            pl.BlockSpec((nclass, 1), lambda i: (0, 0)),
        ],
        out_specs=[
            pl.BlockSpec((br, nclass), lambda i: (i, 0)),
            pl.BlockSpec((br, 1), lambda i: (i, 0)),
            pl.BlockSpec((br, 1), lambda i: (i, 0)),
            pl.BlockSpec((br, 1), lambda i: (i, 0)),
            pl.BlockSpec((br, 1), lambda i: (i, 0)),
        ],
        out_shape=[
            jax.ShapeDtypeStruct((n, nclass), f32),
            jax.ShapeDtypeStruct((n, 1), f32),
            jax.ShapeDtypeStruct((n, 1), f32),
            jax.ShapeDtypeStruct((n, 1), f32),
            jax.ShapeDtypeStruct((n, 1), f32),
        ],
    )(adj, h_all, u1, p1, v1T, q1T, W_out, a2s, a2d)

    v2T = v2c.reshape(1, n)
    q2T = q2c.reshape(1, n)

    out = pl.pallas_call(
        functools.partial(_layer2_kernel, nblk),
        grid=(nblk,),
        in_specs=[
            pl.BlockSpec((br, n), lambda i: (i, 0)),
            pl.BlockSpec((n, nclass), lambda i: (0, 0)),
            pl.BlockSpec((br, 1), lambda i: (i, 0)),
            pl.BlockSpec((br, 1), lambda i: (i, 0)),
            pl.BlockSpec((1, n), lambda i: (0, 0)),
            pl.BlockSpec((1, n), lambda i: (0, 0)),
            pl.BlockSpec((nv, br), lambda i: (0, i)),
        ],
        out_specs=pl.BlockSpec((nv, nclass), lambda i: (0, 0)),
        out_shape=jax.ShapeDtypeStruct((nv, nclass), f32),
    )(adj, h2, u2, p2, v2T, q2T, PvT)
    return out


# speedup vs baseline: 5446.9579x; 1.2235x over previous
import functools

import jax
import jax.numpy as jnp
from jax.experimental import pallas as pl

_ALPHA = 0.2
_BR = 512


def _elu(z):
    return jnp.where(z > 0, z, jnp.exp(jnp.minimum(z, 0.0)) - 1.0)


def _prep_kernel(nheads, nhid,
                 x_ref, xT_ref, Wall_ref, WallT_ref, Asrc_ref, AdstT_ref,
                 haug_ref, u_ref, p_ref, vT_ref, qT_ref):
    h = jnp.dot(x_ref[...], Wall_ref[...], preferred_element_type=jnp.float32)
    ones = jnp.ones((h.shape[0], 1), jnp.float32)
    for hd in range(nheads):
        haug_ref[hd] = jnp.concatenate(
            [h[:, hd * nhid:(hd + 1) * nhid], ones], axis=1)
    hT = jnp.dot(WallT_ref[...], xT_ref[...], preferred_element_type=jnp.float32)
    ls = jnp.dot(h, Asrc_ref[...], preferred_element_type=jnp.float32)
    u_ref[...] = jnp.exp(-ls)
    p_ref[...] = jnp.exp(-_ALPHA * ls)
    ldT = jnp.dot(AdstT_ref[...], hT, preferred_element_type=jnp.float32)
    vT_ref[...] = jnp.exp(-ldT)
    qT_ref[...] = jnp.exp(-_ALPHA * ldT)


def _layer1_kernel(nheads, nhid, nclass,
                   adj_ref, haug_ref, u_ref, p_ref, vT_ref, qT_ref,
                   Wout_ref, a2s_ref, a2d_ref,
                   h2aug_ref, u2_ref, p2_ref, v2_ref, q2_ref):
    adj = adj_ref[...]
    outs = []
    for hd in range(nheads):
        uc = u_ref[:, hd:hd + 1]
        pc = p_ref[:, hd:hd + 1]
        vr = vT_ref[hd:hd + 1, :]
        qr = qT_ref[hd:hd + 1, :]
        e = jnp.minimum(uc * vr, pc * qr) * adj
        hp = jnp.dot(e, haug_ref[hd], preferred_element_type=jnp.float32)
        outs.append(_elu(hp[:, :nhid] / hp[:, nhid:nhid + 1]))
    x1 = jnp.concatenate(outs, axis=1)
    h2 = jnp.dot(x1, Wout_ref[...], preferred_element_type=jnp.float32)
    ones = jnp.ones((h2.shape[0], 1), jnp.float32)
    h2aug_ref[...] = jnp.concatenate([h2, ones], axis=1)
    ls2 = jnp.dot(h2, a2s_ref[...], preferred_element_type=jnp.float32)
    ld2 = jnp.dot(h2, a2d_ref[...], preferred_element_type=jnp.float32)
    u2_ref[...] = jnp.exp(-ls2)
    p2_ref[...] = jnp.exp(-_ALPHA * ls2)
    v2_ref[...] = jnp.exp(-ld2)
    q2_ref[...] = jnp.exp(-_ALPHA * ld2)


def _layer2_kernel(nblk, nclass,
                   adj_ref, h2aug_ref, u2_ref, p2_ref, v2T_ref, q2T_ref,
                   PvT_ref, out_ref):
    i = pl.program_id(0)
    adj = adj_ref[...]
    e = jnp.minimum(u2_ref[...] * v2T_ref[...],
                    p2_ref[...] * q2T_ref[...]) * adj
    hp = jnp.dot(e, h2aug_ref[...], preferred_element_type=jnp.float32)
    x2 = _elu(hp[:, :nclass] / hp[:, nclass:nclass + 1])
    contrib = jnp.dot(PvT_ref[...], x2, preferred_element_type=jnp.float32)

    @pl.when(i == 0)
    def _():
        out_ref[...] = contrib

    @pl.when(i > 0)
    def _():
        out_ref[...] += contrib

    @pl.when(i == nblk - 1)
    def _():
        z = out_ref[...]
        m = jnp.max(z, axis=1, keepdims=True)
        zs = z - m
        out_ref[...] = zs - jnp.log(jnp.sum(jnp.exp(zs), axis=1, keepdims=True))


def kernel(x, adj, PvT, W_heads, a_heads, W_out, a_out):
    f32 = jnp.float32
    n, nfeat = x.shape
    nheads, _, nhid = W_heads.shape
    nclass = W_out.shape[1]
    nv = PvT.shape[0]
    fcat = nheads * nhid
    br = _BR if n % _BR == 0 else n
    nblk = n // br

    Wall = jnp.transpose(W_heads, (1, 0, 2)).reshape(nfeat, fcat)
    WallT = Wall.T
    a_src = a_heads[:, 0, :nhid]
    a_dst = a_heads[:, 0, nhid:]
    eye = jnp.eye(nheads, dtype=f32)
    Asrc = (eye[:, None, :] * a_src[:, :, None]).reshape(fcat, nheads)
    AdstT = (eye[:, :, None] * a_dst[None, :, :]).reshape(nheads, fcat)
    a2s = a_out[0, :nclass].reshape(nclass, 1)
    a2d = a_out[0, nclass:].reshape(nclass, 1)
    xT = x.T

    haug, u1, p1, v1T, q1T = pl.pallas_call(
        functools.partial(_prep_kernel, nheads, nhid),
        out_shape=[
            jax.ShapeDtypeStruct((nheads, n, nhid + 1), f32),
            jax.ShapeDtypeStruct((n, nheads), f32),
            jax.ShapeDtypeStruct((n, nheads), f32),
            jax.ShapeDtypeStruct((nheads, n), f32),
            jax.ShapeDtypeStruct((nheads, n), f32),
        ],
    )(x, xT, Wall, WallT, Asrc, AdstT)

    h2aug, u2, p2, v2c, q2c = pl.pallas_call(
        functools.partial(_layer1_kernel, nheads, nhid, nclass),
        grid=(nblk,),
        in_specs=[
            pl.BlockSpec((br, n), lambda i: (i, 0)),
            pl.BlockSpec((nheads, n, nhid + 1), lambda i: (0, 0, 0)),
            pl.BlockSpec((br, nheads), lambda i: (i, 0)),
            pl.BlockSpec((br, nheads), lambda i: (i, 0)),
            pl.BlockSpec((nheads, n), lambda i: (0, 0)),
            pl.BlockSpec((nheads, n), lambda i: (0, 0)),
            pl.BlockSpec((fcat, nclass), lambda i: (0, 0)),
            pl.BlockSpec((nclass, 1), lambda i: (0, 0)),
            pl.BlockSpec((nclass, 1), lambda i: (0, 0)),
        ],
        out_specs=[
            pl.BlockSpec((br, nclass + 1), lambda i: (i, 0)),
            pl.BlockSpec((br, 1), lambda i: (i, 0)),
            pl.BlockSpec((br, 1), lambda i: (i, 0)),
            pl.BlockSpec((br, 1), lambda i: (i, 0)),
            pl.BlockSpec((br, 1), lambda i: (i, 0)),
        ],
        out_shape=[
            jax.ShapeDtypeStruct((n, nclass + 1), f32),
            jax.ShapeDtypeStruct((n, 1), f32),
            jax.ShapeDtypeStruct((n, 1), f32),
            jax.ShapeDtypeStruct((n, 1), f32),
            jax.ShapeDtypeStruct((n, 1), f32),
        ],
    )(adj, haug, u1, p1, v1T, q1T, W_out, a2s, a2d)

    v2T = v2c.reshape(1, n)
    q2T = q2c.reshape(1, n)

    out = pl.pallas_call(
        functools.partial(_layer2_kernel, nblk, nclass),
        grid=(nblk,),
        in_specs=[
            pl.BlockSpec((br, n), lambda i: (i, 0)),
            pl.BlockSpec((n, nclass + 1), lambda i: (0, 0)),
            pl.BlockSpec((br, 1), lambda i: (i, 0)),
            pl.BlockSpec((br, 1), lambda i: (i, 0)),
            pl.BlockSpec((1, n), lambda i: (0, 0)),
            pl.BlockSpec((1, n), lambda i: (0, 0)),
            pl.BlockSpec((nv, br), lambda i: (0, i)),
        ],
        out_specs=pl.BlockSpec((nv, nclass), lambda i: (0, 0)),
        out_shape=jax.ShapeDtypeStruct((nv, nclass), f32),
    )(adj, h2aug, u2, p2, v2T, q2T, PvT)
    return out


# speedup vs baseline: 6149.0863x; 1.1289x over previous
import functools

import jax
import jax.numpy as jnp
from jax.experimental import pallas as pl
from jax.experimental.pallas import tpu as pltpu

_ALPHA = 0.2
_BR = 512


def _elu(z):
    return jnp.where(z > 0, z, jnp.exp(jnp.minimum(z, 0.0)) - 1.0)


def _fused1_kernel(nheads, nhid, br,
                   x_ref, xT_ref, Wall_ref, WallT_ref, Asrc_ref, AdstT_ref,
                   adj_ref, Wout_ref, a2s_ref, a2d_ref,
                   h2aug_ref, u2_ref, p2_ref, v2_ref, q2_ref,
                   haug_s, u_s, p_s, vT_s, qT_s):
    i = pl.program_id(0)

    @pl.when(i == 0)
    def _():
        h = jnp.dot(x_ref[...], Wall_ref[...],
                    preferred_element_type=jnp.float32)
        ones = jnp.ones((h.shape[0], 1), jnp.float32)
        for hd in range(nheads):
            haug_s[hd] = jnp.concatenate(
                [h[:, hd * nhid:(hd + 1) * nhid], ones], axis=1)
        hT = jnp.dot(WallT_ref[...], xT_ref[...],
                     preferred_element_type=jnp.float32)
        ls = jnp.dot(h, Asrc_ref[...], preferred_element_type=jnp.float32)
        u_s[...] = jnp.exp(-ls)
        p_s[...] = jnp.exp(-_ALPHA * ls)
        ldT = jnp.dot(AdstT_ref[...], hT, preferred_element_type=jnp.float32)
        vT_s[...] = jnp.exp(-ldT)
        qT_s[...] = jnp.exp(-_ALPHA * ldT)

    adj = adj_ref[...]
    row0 = i * br
    outs = []
    for hd in range(nheads):
        uc = u_s[pl.ds(row0, br), hd:hd + 1]
        pc = p_s[pl.ds(row0, br), hd:hd + 1]
        vr = vT_s[hd:hd + 1, :]
        qr = qT_s[hd:hd + 1, :]
        e = jnp.minimum(uc * vr, pc * qr) * adj
        hp = jnp.dot(e, haug_s[hd], preferred_element_type=jnp.float32)
        outs.append(_elu(hp[:, :nhid] / hp[:, nhid:nhid + 1]))
    x1 = jnp.concatenate(outs, axis=1)
    h2 = jnp.dot(x1, Wout_ref[...], preferred_element_type=jnp.float32)
    ones = jnp.ones((h2.shape[0], 1), jnp.float32)
    h2aug_ref[...] = jnp.concatenate([h2, ones], axis=1)
    ls2 = jnp.dot(h2, a2s_ref[...], preferred_element_type=jnp.float32)
    ld2 = jnp.dot(h2, a2d_ref[...], preferred_element_type=jnp.float32)
    u2_ref[...] = jnp.exp(-ls2)
    p2_ref[...] = jnp.exp(-_ALPHA * ls2)
    v2_ref[...] = jnp.exp(-ld2)
    q2_ref[...] = jnp.exp(-_ALPHA * ld2)


def _layer2_kernel(nblk, nclass,
                   adj_ref, h2aug_ref, u2_ref, p2_ref, v2T_ref, q2T_ref,
                   PvT_ref, out_ref):
    i = pl.program_id(0)
    adj = adj_ref[...]
    e = jnp.minimum(u2_ref[...] * v2T_ref[...],
                    p2_ref[...] * q2T_ref[...]) * adj
    hp = jnp.dot(e, h2aug_ref[...], preferred_element_type=jnp.float32)
    x2 = _elu(hp[:, :nclass] / hp[:, nclass:nclass + 1])
    contrib = jnp.dot(PvT_ref[...], x2, preferred_element_type=jnp.float32)

    @pl.when(i == 0)
    def _():
        out_ref[...] = contrib

    @pl.when(i > 0)
    def _():
        out_ref[...] += contrib

    @pl.when(i == nblk - 1)
    def _():
        z = out_ref[...]
        m = jnp.max(z, axis=1, keepdims=True)
        zs = z - m
        out_ref[...] = zs - jnp.log(jnp.sum(jnp.exp(zs), axis=1, keepdims=True))


def kernel(x, adj, PvT, W_heads, a_heads, W_out, a_out):
    f32 = jnp.float32
    n, nfeat = x.shape
    nheads, _, nhid = W_heads.shape
    nclass = W_out.shape[1]
    nv = PvT.shape[0]
    fcat = nheads * nhid
    br = _BR if n % _BR == 0 else n
    nblk = n // br

    Wall = jnp.transpose(W_heads, (1, 0, 2)).reshape(nfeat, fcat)
    WallT = Wall.T
    a_src = a_heads[:, 0, :nhid]
    a_dst = a_heads[:, 0, nhid:]
    eye = jnp.eye(nheads, dtype=f32)
    Asrc = (eye[:, None, :] * a_src[:, :, None]).reshape(fcat, nheads)
    AdstT = (eye[:, :, None] * a_dst[None, :, :]).reshape(nheads, fcat)
    a2s = a_out[0, :nclass].reshape(nclass, 1)
    a2d = a_out[0, nclass:].reshape(nclass, 1)
    xT = x.T

    h2aug, u2, p2, v2c, q2c = pl.pallas_call(
        functools.partial(_fused1_kernel, nheads, nhid, br),
        grid=(nblk,),
        in_specs=[
            pl.BlockSpec((n, nfeat), lambda i: (0, 0)),
            pl.BlockSpec((nfeat, n), lambda i: (0, 0)),
            pl.BlockSpec((nfeat, fcat), lambda i: (0, 0)),
            pl.BlockSpec((fcat, nfeat), lambda i: (0, 0)),
            pl.BlockSpec((fcat, nheads), lambda i: (0, 0)),
            pl.BlockSpec((nheads, fcat), lambda i: (0, 0)),
            pl.BlockSpec((br, n), lambda i: (i, 0)),
            pl.BlockSpec((fcat, nclass), lambda i: (0, 0)),
            pl.BlockSpec((nclass, 1), lambda i: (0, 0)),
            pl.BlockSpec((nclass, 1), lambda i: (0, 0)),
        ],
        out_specs=[
            pl.BlockSpec((br, nclass + 1), lambda i: (i, 0)),
            pl.BlockSpec((br, 1), lambda i: (i, 0)),
            pl.BlockSpec((br, 1), lambda i: (i, 0)),
            pl.BlockSpec((br, 1), lambda i: (i, 0)),
            pl.BlockSpec((br, 1), lambda i: (i, 0)),
        ],
        out_shape=[
            jax.ShapeDtypeStruct((n, nclass + 1), f32),
            jax.ShapeDtypeStruct((n, 1), f32),
            jax.ShapeDtypeStruct((n, 1), f32),
            jax.ShapeDtypeStruct((n, 1), f32),
            jax.ShapeDtypeStruct((n, 1), f32),
        ],
        scratch_shapes=[
            pltpu.VMEM((nheads, n, nhid + 1), f32),
            pltpu.VMEM((n, nheads), f32),
            pltpu.VMEM((n, nheads), f32),
            pltpu.VMEM((nheads, n), f32),
            pltpu.VMEM((nheads, n), f32),
        ],
    )(x, xT, Wall, WallT, Asrc, AdstT, adj, W_out, a2s, a2d)

    v2T = v2c.reshape(1, n)
    q2T = q2c.reshape(1, n)

    out = pl.pallas_call(
        functools.partial(_layer2_kernel, nblk, nclass),
        grid=(nblk,),
        in_specs=[
            pl.BlockSpec((br, n), lambda i: (i, 0)),
            pl.BlockSpec((n, nclass + 1), lambda i: (0, 0)),
            pl.BlockSpec((br, 1), lambda i: (i, 0)),
            pl.BlockSpec((br, 1), lambda i: (i, 0)),
            pl.BlockSpec((1, n), lambda i: (0, 0)),
            pl.BlockSpec((1, n), lambda i: (0, 0)),
            pl.BlockSpec((nv, br), lambda i: (0, i)),
        ],
        out_specs=pl.BlockSpec((nv, nclass), lambda i: (0, 0)),
        out_shape=jax.ShapeDtypeStruct((nv, nclass), f32),
    )(adj, h2aug, u2, p2, v2T, q2T, PvT)
    return out


# speedup vs baseline: 6894.2120x; 1.1212x over previous
import functools

import jax
import jax.numpy as jnp
from jax.experimental import pallas as pl
from jax.experimental.pallas import tpu as pltpu

_ALPHA = 0.2
_BR = 512


def _elu(z):
    return jnp.where(z > 0, z, jnp.exp(jnp.minimum(z, 0.0)) - 1.0)


def _gat_kernel(nheads, nhid, nclass, br, nblk,
                x_ref, xT_ref, Wall_ref, WallT_ref, Asrc_ref, AdstT_ref,
                adj_ref, Wout_ref, a2s_ref, a2d_ref, PvT_ref,
                out_ref,
                haug_s, u_s, p_s, vT_s, qT_s,
                h2aug_s, u2_s, p2_s, v2_s, q2_s, v2T_s, q2T_s):
    i = pl.program_id(0)
    blk = jax.lax.rem(i, nblk)
    row0 = blk * br

    @pl.when(i == 0)
    def _():
        h = jnp.dot(x_ref[...], Wall_ref[...],
                    preferred_element_type=jnp.float32)
        ones = jnp.ones((h.shape[0], 1), jnp.float32)
        for hd in range(nheads):
            haug_s[hd] = jnp.concatenate(
                [h[:, hd * nhid:(hd + 1) * nhid], ones], axis=1)
        hT = jnp.dot(WallT_ref[...], xT_ref[...],
                     preferred_element_type=jnp.float32)
        ls = jnp.dot(h, Asrc_ref[...], preferred_element_type=jnp.float32)
        u_s[...] = jnp.exp(-ls)
        p_s[...] = jnp.exp(-_ALPHA * ls)
        ldT = jnp.dot(AdstT_ref[...], hT, preferred_element_type=jnp.float32)
        vT_s[...] = jnp.exp(-ldT)
        qT_s[...] = jnp.exp(-_ALPHA * ldT)

    adj = adj_ref[...]

    @pl.when(i < nblk)
    def _():
        outs = []
        for hd in range(nheads):
            uc = u_s[pl.ds(row0, br), hd:hd + 1]
            pc = p_s[pl.ds(row0, br), hd:hd + 1]
            vr = vT_s[hd:hd + 1, :]
            qr = qT_s[hd:hd + 1, :]
            e = jnp.minimum(uc * vr, pc * qr) * adj
            hp = jnp.dot(e, haug_s[hd], preferred_element_type=jnp.float32)
            outs.append(_elu(hp[:, :nhid] / hp[:, nhid:nhid + 1]))
        x1 = jnp.concatenate(outs, axis=1)
        h2 = jnp.dot(x1, Wout_ref[...], preferred_element_type=jnp.float32)
        ones = jnp.ones((h2.shape[0], 1), jnp.float32)
        h2aug_s[pl.ds(row0, br), :] = jnp.concatenate([h2, ones], axis=1)
        ls2 = jnp.dot(h2, a2s_ref[...], preferred_element_type=jnp.float32)
        ld2 = jnp.dot(h2, a2d_ref[...], preferred_element_type=jnp.float32)
        u2_s[pl.ds(row0, br), :] = jnp.exp(-ls2)
        p2_s[pl.ds(row0, br), :] = jnp.exp(-_ALPHA * ls2)
        v2_s[pl.ds(row0, br), :] = jnp.exp(-ld2)
        q2_s[pl.ds(row0, br), :] = jnp.exp(-_ALPHA * ld2)

    @pl.when(i == nblk)
    def _():
        v2T_s[...] = jnp.transpose(v2_s[...], (1, 0))
        q2T_s[...] = jnp.transpose(q2_s[...], (1, 0))

    @pl.when(i >= nblk)
    def _():
        e = jnp.minimum(u2_s[pl.ds(row0, br), :] * v2T_s[...],
                        p2_s[pl.ds(row0, br), :] * q2T_s[...]) * adj
        hp = jnp.dot(e, h2aug_s[...], preferred_element_type=jnp.float32)
        x2 = _elu(hp[:, :nclass] / hp[:, nclass:nclass + 1])
        contrib = jnp.dot(PvT_ref[:, pl.ds(row0, br)], x2,
                          preferred_element_type=jnp.float32)

        @pl.when(i == nblk)
        def _():
            out_ref[...] = contrib

        @pl.when(i > nblk)
        def _():
            out_ref[...] += contrib

        @pl.when(i == 2 * nblk - 1)
        def _():
            z = out_ref[...]
            m = jnp.max(z, axis=1, keepdims=True)
            zs = z - m
            out_ref[...] = zs - jnp.log(
                jnp.sum(jnp.exp(zs), axis=1, keepdims=True))


def kernel(x, adj, PvT, W_heads, a_heads, W_out, a_out):
    f32 = jnp.float32
    n, nfeat = x.shape
    nheads, _, nhid = W_heads.shape
    nclass = W_out.shape[1]
    nv = PvT.shape[0]
    fcat = nheads * nhid
    br = _BR if n % _BR == 0 else n
    nblk = n // br

    Wall = jnp.transpose(W_heads, (1, 0, 2)).reshape(nfeat, fcat)
    WallT = Wall.T
    a_src = a_heads[:, 0, :nhid]
    a_dst = a_heads[:, 0, nhid:]
    eye = jnp.eye(nheads, dtype=f32)
    Asrc = (eye[:, None, :] * a_src[:, :, None]).reshape(fcat, nheads)
    AdstT = (eye[:, :, None] * a_dst[None, :, :]).reshape(nheads, fcat)
    a2s = a_out[0, :nclass].reshape(nclass, 1)
    a2d = a_out[0, nclass:].reshape(nclass, 1)
    xT = x.T

    out = pl.pallas_call(
        functools.partial(_gat_kernel, nheads, nhid, nclass, br, nblk),
        grid=(2 * nblk,),
        in_specs=[
            pl.BlockSpec((n, nfeat), lambda i: (0, 0)),
            pl.BlockSpec((nfeat, n), lambda i: (0, 0)),
            pl.BlockSpec((nfeat, fcat), lambda i: (0, 0)),
            pl.BlockSpec((fcat, nfeat), lambda i: (0, 0)),
            pl.BlockSpec((fcat, nheads), lambda i: (0, 0)),
            pl.BlockSpec((nheads, fcat), lambda i: (0, 0)),
            pl.BlockSpec((br, n), lambda i: (jax.lax.rem(i, nblk), 0)),
            pl.BlockSpec((fcat, nclass), lambda i: (0, 0)),
            pl.BlockSpec((nclass, 1), lambda i: (0, 0)),
            pl.BlockSpec((nclass, 1), lambda i: (0, 0)),
            pl.BlockSpec((nv, n), lambda i: (0, 0)),
        ],
        out_specs=pl.BlockSpec((nv, nclass), lambda i: (0, 0)),
        out_shape=jax.ShapeDtypeStruct((nv, nclass), f32),
        scratch_shapes=[
            pltpu.VMEM((nheads, n, nhid + 1), f32),
            pltpu.VMEM((n, nheads), f32),
            pltpu.VMEM((n, nheads), f32),
            pltpu.VMEM((nheads, n), f32),
            pltpu.VMEM((nheads, n), f32),
            pltpu.VMEM((n, nclass + 1), f32),
            pltpu.VMEM((n, 1), f32),
            pltpu.VMEM((n, 1), f32),
            pltpu.VMEM((n, 1), f32),
            pltpu.VMEM((n, 1), f32),
            pltpu.VMEM((1, n), f32),
            pltpu.VMEM((1, n), f32),
        ],
    )(x, xT, Wall, WallT, Asrc, AdstT, adj, W_out, a2s, a2d, PvT)
    return out


# speedup vs baseline: 7170.7901x; 1.0401x over previous
import functools

import jax
import jax.numpy as jnp
from jax.experimental import pallas as pl
from jax.experimental.pallas import tpu as pltpu

_ALPHA = 0.2
_BR = 512


def _elu(z):
    return jnp.where(z > 0, z, jnp.exp(jnp.minimum(z, 0.0)) - 1.0)


def _gat_kernel(nheads, nhid, nclass, br, nblk,
                x_ref, Wall_ref, Asrc_ref, Adst_ref,
                adj_ref, Wout_ref, a2s_ref, a2d_ref, PvT_ref,
                out_ref,
                haug_s, u_s, p_s, vT_s, qT_s,
                h2aug_s, u2_s, p2_s, v2T_s, q2T_s):
    i = pl.program_id(0)
    blk = jax.lax.rem(i, nblk)
    row0 = blk * br

    @pl.when(i == 0)
    def _():
        h = jnp.dot(x_ref[...], Wall_ref[...],
                    preferred_element_type=jnp.float32)
        ones = jnp.ones((h.shape[0], 1), jnp.float32)
        for hd in range(nheads):
            haug_s[hd] = jnp.concatenate(
                [h[:, hd * nhid:(hd + 1) * nhid], ones], axis=1)
        ls = jnp.dot(h, Asrc_ref[...], preferred_element_type=jnp.float32)
        u_s[...] = jnp.exp(-ls)
        p_s[...] = jnp.exp(-_ALPHA * ls)
        ld = jnp.dot(h, Adst_ref[...], preferred_element_type=jnp.float32)
        ldT = jnp.transpose(ld, (1, 0))
        vT_s[...] = jnp.exp(-ldT)
        qT_s[...] = jnp.exp(-_ALPHA * ldT)

    adj = adj_ref[pl.ds(row0, br), :]

    @pl.when(i < nblk)
    def _():
        outs = []
        for hd in range(nheads):
            uc = u_s[pl.ds(row0, br), hd:hd + 1]
            pc = p_s[pl.ds(row0, br), hd:hd + 1]
            vr = vT_s[hd:hd + 1, :]
            qr = qT_s[hd:hd + 1, :]
            e = jnp.minimum(uc * vr, pc * qr) * adj
            hp = jnp.dot(e, haug_s[hd], preferred_element_type=jnp.float32)
            outs.append(_elu(hp[:, :nhid] / hp[:, nhid:nhid + 1]))
        x1 = jnp.concatenate(outs, axis=1)
        h2 = jnp.dot(x1, Wout_ref[...], preferred_element_type=jnp.float32)
        ones = jnp.ones((h2.shape[0], 1), jnp.float32)
        h2aug_s[pl.ds(row0, br), :] = jnp.concatenate([h2, ones], axis=1)
        ls2 = jnp.dot(h2, a2s_ref[...], preferred_element_type=jnp.float32)
        ld2 = jnp.dot(h2, a2d_ref[...], preferred_element_type=jnp.float32)
        u2_s[pl.ds(row0, br), :] = jnp.exp(-ls2)
        p2_s[pl.ds(row0, br), :] = jnp.exp(-_ALPHA * ls2)
        ld2T = jnp.transpose(ld2, (1, 0))
        v2T_s[0:1, pl.ds(row0, br)] = jnp.exp(-ld2T)
        q2T_s[0:1, pl.ds(row0, br)] = jnp.exp(-_ALPHA * ld2T)

    @pl.when(i >= nblk)
    def _():
        e = jnp.minimum(u2_s[pl.ds(row0, br), :] * v2T_s[...],
                        p2_s[pl.ds(row0, br), :] * q2T_s[...]) * adj
        hp = jnp.dot(e, h2aug_s[...], preferred_element_type=jnp.float32)
        x2 = _elu(hp[:, :nclass] / hp[:, nclass:nclass + 1])
        contrib = jnp.dot(PvT_ref[:, pl.ds(row0, br)], x2,
                          preferred_element_type=jnp.float32)

        @pl.when(i == nblk)
        def _():
            out_ref[...] = contrib

        @pl.when(i > nblk)
        def _():
            out_ref[...] += contrib

        @pl.when(i == 2 * nblk - 1)
        def _():
            z = out_ref[...]
            m = jnp.max(z, axis=1, keepdims=True)
            zs = z - m
            out_ref[...] = zs - jnp.log(
                jnp.sum(jnp.exp(zs), axis=1, keepdims=True))


def kernel(x, adj, PvT, W_heads, a_heads, W_out, a_out):
    f32 = jnp.float32
    n, nfeat = x.shape
    nheads, _, nhid = W_heads.shape
    nclass = W_out.shape[1]
    nv = PvT.shape[0]
    fcat = nheads * nhid
    br = _BR if n % _BR == 0 else n
    nblk = n // br

    Wall = jnp.transpose(W_heads, (1, 0, 2)).reshape(nfeat, fcat)
    a_src = a_heads[:, 0, :nhid]
    a_dst = a_heads[:, 0, nhid:]
    eye = jnp.eye(nheads, dtype=f32)
    Asrc = (eye[:, None, :] * a_src[:, :, None]).reshape(fcat, nheads)
    Adst = (eye[:, None, :] * a_dst[:, :, None]).reshape(fcat, nheads)
    a2s = a_out[0, :nclass].reshape(nclass, 1)
    a2d = a_out[0, nclass:].reshape(nclass, 1)

    out = pl.pallas_call(
        functools.partial(_gat_kernel, nheads, nhid, nclass, br, nblk),
        grid=(2 * nblk,),
        in_specs=[
            pl.BlockSpec((n, nfeat), lambda i: (0, 0)),
            pl.BlockSpec((nfeat, fcat), lambda i: (0, 0)),
            pl.BlockSpec((fcat, nheads), lambda i: (0, 0)),
            pl.BlockSpec((fcat, nheads), lambda i: (0, 0)),
            pl.BlockSpec((n, n), lambda i: (0, 0)),
            pl.BlockSpec((fcat, nclass), lambda i: (0, 0)),
            pl.BlockSpec((nclass, 1), lambda i: (0, 0)),
            pl.BlockSpec((nclass, 1), lambda i: (0, 0)),
            pl.BlockSpec((nv, n), lambda i: (0, 0)),
        ],
        out_specs=pl.BlockSpec((nv, nclass), lambda i: (0, 0)),
        out_shape=jax.ShapeDtypeStruct((nv, nclass), f32),
        scratch_shapes=[
            pltpu.VMEM((nheads, n, nhid + 1), f32),
            pltpu.VMEM((n, nheads), f32),
            pltpu.VMEM((n, nheads), f32),
            pltpu.VMEM((nheads, n), f32),
            pltpu.VMEM((nheads, n), f32),
            pltpu.VMEM((n, nclass + 1), f32),
            pltpu.VMEM((n, 1), f32),
            pltpu.VMEM((n, 1), f32),
            pltpu.VMEM((1, n), f32),
            pltpu.VMEM((1, n), f32),
        ],
    )(x, Wall, Asrc, Adst, adj, W_out, a2s, a2d, PvT)
    return out


# speedup vs baseline: 8116.9669x; 1.1319x over previous
import functools

import jax
import jax.numpy as jnp
from jax.experimental import pallas as pl
from jax.experimental.pallas import tpu as pltpu

_ALPHA = 0.2
_BR = 512


def _elu(z):
    return jnp.where(z > 0, z, jnp.exp(jnp.minimum(z, 0.0)) - 1.0)


def _gat_kernel(nheads, nhid, nclass, br, nblk,
                x_ref, Wall_ref, Asrc_ref, Adst_ref,
                adj_ref, Wout_ref, a2s_ref, a2d_ref, PvT_ref,
                out_ref,
                haug_s, u_s, p_s, vT_s, qT_s, adj16_s,
                h2aug_s, u2_s, p2_s, v2T_s, q2T_s):
    bf16 = jnp.bfloat16
    i = pl.program_id(0)
    blk = jax.lax.rem(i, nblk)
    row0 = blk * br

    @pl.when(i == 0)
    def _():
        h = jnp.dot(x_ref[...], Wall_ref[...],
                    preferred_element_type=jnp.float32)
        ones = jnp.ones((h.shape[0], 1), jnp.float32)
        for hd in range(nheads):
            haug_s[hd] = jnp.concatenate(
                [h[:, hd * nhid:(hd + 1) * nhid], ones], axis=1).astype(bf16)
        ls = jnp.dot(h, Asrc_ref[...], preferred_element_type=jnp.float32)
        u_s[...] = jnp.exp(-ls).astype(bf16)
        p_s[...] = jnp.exp(-_ALPHA * ls).astype(bf16)
        ld = jnp.dot(h, Adst_ref[...], preferred_element_type=jnp.float32)
        ldT = jnp.transpose(ld, (1, 0))
        vT_s[...] = jnp.exp(-ldT).astype(bf16)
        qT_s[...] = jnp.exp(-_ALPHA * ldT).astype(bf16)

    @pl.when(i < nblk)
    def _():
        adj = adj_ref[pl.ds(row0, br), :].astype(bf16)
        adj16_s[pl.ds(row0, br), :] = adj
        outs = []
        for hd in range(nheads):
            uc = u_s[pl.ds(row0, br), hd:hd + 1]
            pc = p_s[pl.ds(row0, br), hd:hd + 1]
            vr = vT_s[hd:hd + 1, :]
            qr = qT_s[hd:hd + 1, :]
            e = jnp.minimum(uc * vr, pc * qr) * adj
            hp = jnp.dot(e, haug_s[hd], preferred_element_type=jnp.float32)
            outs.append(_elu(hp[:, :nhid] / hp[:, nhid:nhid + 1]))
        x1 = jnp.concatenate(outs, axis=1)
        h2 = jnp.dot(x1, Wout_ref[...], preferred_element_type=jnp.float32)
        ones = jnp.ones((h2.shape[0], 1), jnp.float32)
        h2aug_s[pl.ds(row0, br), :] = jnp.concatenate(
            [h2, ones], axis=1).astype(bf16)
        ls2 = jnp.dot(h2, a2s_ref[...], preferred_element_type=jnp.float32)
        ld2 = jnp.dot(h2, a2d_ref[...], preferred_element_type=jnp.float32)
        u2_s[pl.ds(row0, br), :] = jnp.exp(-ls2).astype(bf16)
        p2_s[pl.ds(row0, br), :] = jnp.exp(-_ALPHA * ls2).astype(bf16)
        ld2T = jnp.transpose(ld2, (1, 0))
        v2T_s[0:1, pl.ds(row0, br)] = jnp.exp(-ld2T).astype(bf16)
        q2T_s[0:1, pl.ds(row0, br)] = jnp.exp(-_ALPHA * ld2T).astype(bf16)

    @pl.when(i >= nblk)
    def _():
        adj = adj16_s[pl.ds(row0, br), :]
        e = jnp.minimum(u2_s[pl.ds(row0, br), :] * v2T_s[...],
                        p2_s[pl.ds(row0, br), :] * q2T_s[...]) * adj
        hp = jnp.dot(e, h2aug_s[...], preferred_element_type=jnp.float32)
        x2 = _elu(hp[:, :nclass] / hp[:, nclass:nclass + 1])
        contrib = jnp.dot(PvT_ref[:, pl.ds(row0, br)], x2,
                          preferred_element_type=jnp.float32)

        @pl.when(i == nblk)
        def _():
            out_ref[...] = contrib

        @pl.when(i > nblk)
        def _():
            out_ref[...] += contrib

        @pl.when(i == 2 * nblk - 1)
        def _():
            z = out_ref[...]
            m = jnp.max(z, axis=1, keepdims=True)
            zs = z - m
            out_ref[...] = zs - jnp.log(
                jnp.sum(jnp.exp(zs), axis=1, keepdims=True))


def kernel(x, adj, PvT, W_heads, a_heads, W_out, a_out):
    f32 = jnp.float32
    bf16 = jnp.bfloat16
    n, nfeat = x.shape
    nheads, _, nhid = W_heads.shape
    nclass = W_out.shape[1]
    nv = PvT.shape[0]
    fcat = nheads * nhid
    br = _BR if n % _BR == 0 else n
    nblk = n // br

    Wall = jnp.transpose(W_heads, (1, 0, 2)).reshape(nfeat, fcat)
    a_src = a_heads[:, 0, :nhid]
    a_dst = a_heads[:, 0, nhid:]
    eye = jnp.eye(nheads, dtype=f32)
    Asrc = (eye[:, None, :] * a_src[:, :, None]).reshape(fcat, nheads)
    Adst = (eye[:, None, :] * a_dst[:, :, None]).reshape(fcat, nheads)
    a2s = a_out[0, :nclass].reshape(nclass, 1)
    a2d = a_out[0, nclass:].reshape(nclass, 1)

    out = pl.pallas_call(
        functools.partial(_gat_kernel, nheads, nhid, nclass, br, nblk),
        grid=(2 * nblk,),
        in_specs=[
            pl.BlockSpec((n, nfeat), lambda i: (0, 0)),
            pl.BlockSpec((nfeat, fcat), lambda i: (0, 0)),
            pl.BlockSpec((fcat, nheads), lambda i: (0, 0)),
            pl.BlockSpec((fcat, nheads), lambda i: (0, 0)),
            pl.BlockSpec((n, n), lambda i: (0, 0)),
            pl.BlockSpec((fcat, nclass), lambda i: (0, 0)),
            pl.BlockSpec((nclass, 1), lambda i: (0, 0)),
            pl.BlockSpec((nclass, 1), lambda i: (0, 0)),
            pl.BlockSpec((nv, n), lambda i: (0, 0)),
        ],
        out_specs=pl.BlockSpec((nv, nclass), lambda i: (0, 0)),
        out_shape=jax.ShapeDtypeStruct((nv, nclass), f32),
        scratch_shapes=[
            pltpu.VMEM((nheads, n, nhid + 1), bf16),
            pltpu.VMEM((n, nheads), bf16),
            pltpu.VMEM((n, nheads), bf16),
            pltpu.VMEM((nheads, n), bf16),
            pltpu.VMEM((nheads, n), bf16),
            pltpu.VMEM((n, n), bf16),
            pltpu.VMEM((n, nclass + 1), bf16),
            pltpu.VMEM((n, 1), bf16),
            pltpu.VMEM((n, 1), bf16),
            pltpu.VMEM((1, n), bf16),
            pltpu.VMEM((1, n), bf16),
        ],
    )(x, Wall, Asrc, Adst, adj, W_out, a2s, a2d, PvT)
    return out


# speedup vs baseline: 8946.9066x; 1.1022x over previous
import functools

import jax
import jax.numpy as jnp
from jax.experimental import pallas as pl
from jax.experimental.pallas import tpu as pltpu

_ALPHA = 0.2
_BR = 512


def _elu(z):
    return jnp.where(z > 0, z, jnp.exp(jnp.minimum(z, 0.0)) - 1.0)


def _gat_kernel(nheads, nhid, nclass, br, nblk,
                x_ref, Wall_ref, Asrc_ref, Adst_ref,
                adj_ref, Wout_ref, a2s_ref, a2d_ref, PvT_ref,
                out_ref,
                haug_s, r_s, vT_s, qT_s, adj16_s,
                h2aug_s, r2_s, v2T_s, q2T_s):
    bf16 = jnp.bfloat16
    i = pl.program_id(0)
    blk = jax.lax.rem(i, nblk)
    row0 = blk * br

    @pl.when(i == 0)
    def _():
        h = jnp.dot(x_ref[...], Wall_ref[...],
                    preferred_element_type=jnp.float32)
        ones = jnp.ones((h.shape[0], 1), jnp.float32)
        for hd in range(nheads):
            haug_s[hd] = jnp.concatenate(
                [h[:, hd * nhid:(hd + 1) * nhid], ones], axis=1).astype(bf16)
        ls = jnp.dot(h, Asrc_ref[...], preferred_element_type=jnp.float32)
        r_s[...] = jnp.exp((1.0 - _ALPHA) * ls).astype(bf16)
        ld = jnp.dot(h, Adst_ref[...], preferred_element_type=jnp.float32)
        ldT = jnp.transpose(ld, (1, 0))
        vT_s[...] = jnp.exp(-ldT).astype(bf16)
        qT_s[...] = jnp.exp(-_ALPHA * ldT).astype(bf16)

    @pl.when(i < nblk)
    def _():
        adj = adj_ref[...].astype(bf16)
        adj16_s[pl.ds(row0, br), :] = adj
        outs = []
        for hd in range(nheads):
            rc = r_s[pl.ds(row0, br), hd:hd + 1]
            vr = vT_s[hd:hd + 1, :]
            qr = qT_s[hd:hd + 1, :]
            e = jnp.minimum(vr, rc * qr) * adj
            hp = jnp.dot(e, haug_s[hd], preferred_element_type=jnp.float32)
            outs.append(_elu(hp[:, :nhid] / hp[:, nhid:nhid + 1]))
        x1 = jnp.concatenate(outs, axis=1)
        h2 = jnp.dot(x1, Wout_ref[...], preferred_element_type=jnp.float32)
        ones = jnp.ones((h2.shape[0], 1), jnp.float32)
        h2aug_s[pl.ds(row0, br), :] = jnp.concatenate(
            [h2, ones], axis=1).astype(bf16)
        ls2 = jnp.dot(h2, a2s_ref[...], preferred_element_type=jnp.float32)
        ld2 = jnp.dot(h2, a2d_ref[...], preferred_element_type=jnp.float32)
        r2_s[pl.ds(row0, br), :] = jnp.exp((1.0 - _ALPHA) * ls2).astype(bf16)
        ld2T = jnp.transpose(ld2, (1, 0))
        v2T_s[0:1, pl.ds(row0, br)] = jnp.exp(-ld2T).astype(bf16)
        q2T_s[0:1, pl.ds(row0, br)] = jnp.exp(-_ALPHA * ld2T).astype(bf16)

    @pl.when(i >= nblk)
    def _():
        adj = adj16_s[pl.ds(row0, br), :]
        e = jnp.minimum(v2T_s[...],
                        r2_s[pl.ds(row0, br), :] * q2T_s[...]) * adj
        hp = jnp.dot(e, h2aug_s[...], preferred_element_type=jnp.float32)
        x2 = _elu(hp[:, :nclass] / hp[:, nclass:nclass + 1])
        contrib = jnp.dot(PvT_ref[:, pl.ds(row0, br)], x2,
                          preferred_element_type=jnp.float32)

        @pl.when(i == nblk)
        def _():
            out_ref[...] = contrib

        @pl.when(i > nblk)
        def _():
            out_ref[...] += contrib

        @pl.when(i == 2 * nblk - 1)
        def _():
            z = out_ref[...]
            m = jnp.max(z, axis=1, keepdims=True)
            zs = z - m
            out_ref[...] = zs - jnp.log(
                jnp.sum(jnp.exp(zs), axis=1, keepdims=True))


def kernel(x, adj, PvT, W_heads, a_heads, W_out, a_out):
    f32 = jnp.float32
    bf16 = jnp.bfloat16
    n, nfeat = x.shape
    nheads, _, nhid = W_heads.shape
    nclass = W_out.shape[1]
    nv = PvT.shape[0]
    fcat = nheads * nhid
    br = _BR if n % _BR == 0 else n
    nblk = n // br

    Wall = jnp.transpose(W_heads, (1, 0, 2)).reshape(nfeat, fcat)
    a_src = a_heads[:, 0, :nhid]
    a_dst = a_heads[:, 0, nhid:]
    eye = jnp.eye(nheads, dtype=f32)
    Asrc = (eye[:, None, :] * a_src[:, :, None]).reshape(fcat, nheads)
    Adst = (eye[:, None, :] * a_dst[:, :, None]).reshape(fcat, nheads)
    a2s = a_out[0, :nclass].reshape(nclass, 1)
    a2d = a_out[0, nclass:].reshape(nclass, 1)

    out = pl.pallas_call(
        functools.partial(_gat_kernel, nheads, nhid, nclass, br, nblk),
        grid=(2 * nblk,),
        in_specs=[
            pl.BlockSpec((n, nfeat), lambda i: (0, 0)),
            pl.BlockSpec((nfeat, fcat), lambda i: (0, 0)),
            pl.BlockSpec((fcat, nheads), lambda i: (0, 0)),
            pl.BlockSpec((fcat, nheads), lambda i: (0, 0)),
            pl.BlockSpec((br, n), lambda i: (jnp.minimum(i, nblk - 1), 0)),
            pl.BlockSpec((fcat, nclass), lambda i: (0, 0)),
            pl.BlockSpec((nclass, 1), lambda i: (0, 0)),
            pl.BlockSpec((nclass, 1), lambda i: (0, 0)),
            pl.BlockSpec((nv, n), lambda i: (0, 0)),
        ],
        out_specs=pl.BlockSpec((nv, nclass), lambda i: (0, 0)),
        out_shape=jax.ShapeDtypeStruct((nv, nclass), f32),
        scratch_shapes=[
            pltpu.VMEM((nheads, n, nhid + 1), bf16),
            pltpu.VMEM((n, nheads), bf16),
            pltpu.VMEM((nheads, n), bf16),
            pltpu.VMEM((nheads, n), bf16),
            pltpu.VMEM((n, n), bf16),
            pltpu.VMEM((n, nclass + 1), bf16),
            pltpu.VMEM((n, 1), bf16),
            pltpu.VMEM((1, n), bf16),
            pltpu.VMEM((1, n), bf16),
        ],
    )(x, Wall, Asrc, Adst, adj, W_out, a2s, a2d, PvT)
    return out
